# baseline (speedup 1.0000x reference)
import functools

import jax
import jax.numpy as jnp
from jax import lax
from jax.experimental import pallas as pl
from jax.experimental.pallas import tpu as pltpu
from jax.experimental.pallas import tpu_sc as plsc

_B, _L, _T = 16, 32, 32
_D = 2048
_N = _B * _L * _T
_NC, _NS = 2, 16
_NW = _NC * _NS
_RPW = _N // _NW
_CH = 16
_NCHUNK = _RPW // _CH
_LANES = 16


def _sc_embed(tokens_flat, emb_table, pos_table):
    mesh = plsc.VectorSubcoreMesh(core_axis_name="c", subcore_axis_name="s")

    @functools.partial(
        pl.kernel,
        out_type=jax.ShapeDtypeStruct((_N, _D), jnp.float32),
        mesh=mesh,
        scratch_types=[
            pltpu.VMEM((_RPW,), jnp.int32),
            pltpu.VMEM((_CH, _D), jnp.float32),
            pltpu.VMEM((_D,), jnp.float32),
            pltpu.SemaphoreType.DMA,
        ],
    )
    def k(tok_hbm, emb_hbm, pos_hbm, out_hbm, idx_v, buf, pos_v, sem):
        wid = lax.axis_index("s") * _NC + lax.axis_index("c")
        base = wid * _RPW
        pltpu.sync_copy(tok_hbm.at[pl.ds(base, _RPW)], idx_v)

        def chunk_body(c, carry):
            row0 = base + c * _CH
            l = (row0 // _T) % _L
            pltpu.sync_copy(pos_hbm.at[l], pos_v)
            gather = pltpu.async_copy(
                emb_hbm.at[idx_v.at[pl.ds(c * _CH, _CH)]], buf, sem
            )
            gather.wait()

            def col_body(j, carry2):
                pv = pos_v[pl.ds(j * _LANES, _LANES)]

                def row_body(r, carry3):
                    sl = pl.ds(j * _LANES, _LANES)
                    buf[r, sl] = buf[r, sl] + pv
                    return carry3

                return lax.fori_loop(0, _CH, row_body, carry2)

            lax.fori_loop(0, _D // _LANES, col_body, 0)
            pltpu.sync_copy(buf, out_hbm.at[pl.ds(row0, _CH)])
            return carry

        lax.fori_loop(0, _NCHUNK, chunk_body, 0)

    return k(tokens_flat, emb_table, pos_table)


def kernel(observations_tokens, emb_table, pos_table):
    tokens_flat = observations_tokens.reshape(_N).astype(jnp.int32)
    out = _sc_embed(tokens_flat, emb_table, pos_table)
    return out.reshape(_B, _L, _T, _D)


# speedup vs baseline: 4.8298x; 4.8298x over previous
import functools

import jax
import jax.numpy as jnp
from jax import lax
from jax.experimental import pallas as pl
from jax.experimental.pallas import tpu as pltpu
from jax.experimental.pallas import tpu_sc as plsc

_B, _L, _T = 16, 32, 32
_D = 2048
_N = _B * _L * _T
_NC, _NS = 2, 16
_NW = _NC * _NS
_RPW = _N // _NW
_CH = 8
_NCHUNK = _RPW // _CH
_LPW = _RPW // _T
_LANES = 16


def _sc_embed(tokens_flat, emb_table, pos_table):
    mesh = plsc.VectorSubcoreMesh(core_axis_name="c", subcore_axis_name="s")

    @functools.partial(
        pl.kernel,
        out_type=jax.ShapeDtypeStruct((_N, _D), jnp.float32),
        mesh=mesh,
        scratch_types=[
            pltpu.VMEM((_RPW,), jnp.int32),
            pltpu.VMEM((_LPW, _D), jnp.float32),
            pltpu.VMEM((_CH, _D), jnp.float32),
            pltpu.VMEM((_CH, _D), jnp.float32),
            pltpu.VMEM((_CH, _D), jnp.float32),
            pltpu.VMEM((_CH, _D), jnp.float32),
            pltpu.SemaphoreType.DMA,
            pltpu.SemaphoreType.DMA,
            pltpu.SemaphoreType.DMA,
            pltpu.SemaphoreType.DMA,
        ],
    )
    def k(tok_hbm, emb_hbm, pos_hbm, out_hbm, idx_v, pos16,
          gbuf0, gbuf1, sbuf0, sbuf1, gsem0, gsem1, ssem0, ssem1):
        gbufs = (gbuf0, gbuf1)
        sbufs = (sbuf0, sbuf1)
        gsems = (gsem0, gsem1)
        ssems = (ssem0, ssem1)

        wid = lax.axis_index("s") * _NC + lax.axis_index("c")
        base = wid * _RPW
        l0 = (wid % 2) * _LPW
        pltpu.sync_copy(tok_hbm.at[pl.ds(base, _RPW)], idx_v)
        pltpu.sync_copy(pos_hbm.at[pl.ds(l0, _LPW)], pos16)

        for b in range(2):
            pltpu.async_copy(
                emb_hbm.at[idx_v.at[pl.ds(b * _CH, _CH)]], gbufs[b], gsems[b]
            )

        def pair_body(i, carry):
            for b in range(2):
                c = 2 * i + b
                pltpu.make_async_copy(
                    emb_hbm.at[idx_v.at[pl.ds(0, _CH)]], gbufs[b], gsems[b]
                ).wait()

                @pl.when(c >= 2)
                def _():
                    pltpu.make_async_copy(
                        sbufs[b], out_hbm.at[pl.ds(base, _CH)], ssems[b]
                    ).wait()

                lc = c // (_T // _CH)

                @plsc.parallel_loop(0, _D // _LANES, unroll=4)
                def _(j):
                    sl = pl.ds(j * _LANES, _LANES)
                    pv = pos16[lc, sl]
                    for r in range(_CH):
                        sbufs[b][r, sl] = gbufs[b][r, sl] + pv

                pltpu.async_copy(
                    sbufs[b], out_hbm.at[pl.ds(base + c * _CH, _CH)], ssems[b]
                )

                @pl.when(c + 2 < _NCHUNK)
                def _():
                    pltpu.async_copy(
                        emb_hbm.at[idx_v.at[pl.ds((c + 2) * _CH, _CH)]],
                        gbufs[b],
                        gsems[b],
                    )

            return carry

        lax.fori_loop(0, _NCHUNK // 2, pair_body, 0)

        for b in range(2):
            pltpu.make_async_copy(
                sbufs[b], out_hbm.at[pl.ds(base, _CH)], ssems[b]
            ).wait()

    return k(tokens_flat, emb_table, pos_table)


def kernel(observations_tokens, emb_table, pos_table):
    tokens_flat = observations_tokens.reshape(_N).astype(jnp.int32)
    out = _sc_embed(tokens_flat, emb_table, pos_table)
    return out.reshape(_B, _L, _T, _D)
